# baseline (speedup 1.0000x reference)
import functools

import jax
import jax.numpy as jnp
from jax import lax
from jax.experimental import pallas as pl
from jax.experimental.pallas import tpu as pltpu
from jax.experimental.pallas import tpu_sc as plsc

_F32 = jnp.float32


def _make_deg(N, E):
    CH = E // 32
    K = 1000
    assert CH % K == 0 and K % 8 == 0
    nchunks = CH // K
    rows = N // 16 - 1
    assert rows % 8 == 0 and rows * 16 + 16 == N
    mesh = plsc.VectorSubcoreMesh(core_axis_name="c", subcore_axis_name="s")

    @functools.partial(
        pl.kernel, mesh=mesh,
        compiler_params=pltpu.CompilerParams(use_tc_tiling_on_sc=False),
        out_type=jax.ShapeDtypeStruct((2 * N,), _F32),
        scratch_types=[
            pltpu.VMEM((K,), jnp.int32),
            pltpu.VMEM((K,), _F32),
            pltpu.VMEM((rows,), _F32),
            pltpu.VMEM_SHARED((N,), _F32),
        ],
    )
    def deg(dst_hbm, ones_hbm, zeros_hbm, out_hbm, idx_d, ones_v, stage_v, sh_deg):
        c = lax.axis_index("c")
        s = lax.axis_index("s")
        wid = s * 2 + c
        pltpu.sync_copy(ones_hbm, ones_v)
        pltpu.sync_copy(zeros_hbm, stage_v)
        pltpu.sync_copy(stage_v, sh_deg.at[pl.ds(s * rows, rows)])

        @pl.when(s == 15)
        def _():
            pltpu.sync_copy(stage_v.at[pl.ds(0, 16)], sh_deg.at[pl.ds(16 * rows, 16)])

        plsc.subcore_barrier()
        e0 = wid * CH
        for k in range(nchunks):
            pltpu.sync_copy(dst_hbm.at[pl.ds(e0 + k * K, K)], idx_d)
            pltpu.sync_copy(ones_v, sh_deg.at[idx_d], add=True)
        plsc.subcore_barrier()
        pltpu.sync_copy(sh_deg.at[pl.ds(s * rows, rows)], stage_v)
        pltpu.sync_copy(stage_v, out_hbm.at[pl.ds(c * N + s * rows, rows)])

        @pl.when(s == 15)
        def _():
            pltpu.sync_copy(sh_deg.at[pl.ds(16 * rows, 16)], stage_v.at[pl.ds(0, 16)])
            pltpu.sync_copy(stage_v.at[pl.ds(0, 16)],
                            out_hbm.at[pl.ds(c * N + 16 * rows, 16)])

    return deg


def _make_agg(N, E, H):
    CH = E // 16
    K = 200
    NBUF = 4
    assert CH % (NBUF * K) == 0 and K % 8 == 0
    nchunks = CH // K
    rows = N // 16 - 1
    assert rows % 8 == 0 and rows * 16 + 16 == N
    assert NBUF * K >= rows + 16
    mesh = plsc.VectorSubcoreMesh(core_axis_name="c", subcore_axis_name="s")

    @functools.partial(
        pl.kernel, mesh=mesh,
        compiler_params=pltpu.CompilerParams(use_tc_tiling_on_sc=False),
        out_type=jax.ShapeDtypeStruct((2 * N, H), _F32),
        scratch_types=[
            pltpu.VMEM((NBUF * K,), jnp.int32),
            [pltpu.VMEM((K,), jnp.int32)] * NBUF,
            pltpu.VMEM((NBUF * K, H), _F32),
            pltpu.VMEM_SHARED((N, H), _F32),
            [pltpu.SemaphoreType.DMA] * NBUF,
            [pltpu.SemaphoreType.DMA] * NBUF,
        ],
    )
    def agg(g0_hbm, g1_hbm, src_hbm, dst_hbm, zeros_hbm, out_hbm,
            idx_s, idb, rows_v, sh_acc, gsems, ssems):
        c = lax.axis_index("c")
        s = lax.axis_index("s")
        isb = [idx_s.at[pl.ds(i * K, K)] for i in range(NBUF)]
        bufs = [rows_v.at[pl.ds(i * K, K)] for i in range(NBUF)]
        pltpu.sync_copy(zeros_hbm, rows_v.at[pl.ds(0, rows)])
        pltpu.sync_copy(rows_v.at[pl.ds(0, rows)], sh_acc.at[pl.ds(s * rows, rows)])

        @pl.when(s == 15)
        def _():
            pltpu.sync_copy(rows_v.at[pl.ds(0, 16)], sh_acc.at[pl.ds(16 * rows, 16)])

        plsc.subcore_barrier()
        e0 = s * CH

        def edge_loop(g_hbm):
            gh = [None] * NBUF
            sh = [None] * NBUF

            def issue(j, b):
                pltpu.sync_copy(src_hbm.at[pl.ds(e0 + j * K, K)], isb[b])
                pltpu.sync_copy(dst_hbm.at[pl.ds(e0 + j * K, K)], idb[b])
                gh[b] = pltpu.async_copy(g_hbm.at[isb[b]], bufs[b], gsems[b])

            issue(0, 0)
            issue(1, 1)
            for j in range(nchunks):
                b = j % NBUF
                gh[b].wait()
                sh[b] = pltpu.async_copy(bufs[b], sh_acc.at[idb[b]], ssems[b],
                                         add=True)
                if j + 2 < nchunks:
                    b2 = (j + 2) % NBUF
                    if sh[b2] is not None:
                        sh[b2].wait()
                        sh[b2] = None
                    issue(j + 2, b2)
            for b in range(NBUF):
                if sh[b] is not None:
                    sh[b].wait()

        @pl.when(c == 0)
        def _():
            edge_loop(g0_hbm)

        @pl.when(c == 1)
        def _():
            edge_loop(g1_hbm)

        plsc.subcore_barrier()
        pltpu.sync_copy(sh_acc.at[pl.ds(s * rows, rows)], rows_v.at[pl.ds(0, rows)])
        pltpu.sync_copy(rows_v.at[pl.ds(0, rows)],
                        out_hbm.at[pl.ds(c * N + s * rows, rows)])

        @pl.when(s == 15)
        def _():
            pltpu.sync_copy(sh_acc.at[pl.ds(16 * rows, 16)], rows_v.at[pl.ds(0, 16)])
            pltpu.sync_copy(rows_v.at[pl.ds(0, 16)],
                            out_hbm.at[pl.ds(c * N + 16 * rows, 16)])

    return agg


_PREC = jax.lax.Precision.HIGHEST


def _dinv(deg_ref):
    d = deg_ref[:, 0] + deg_ref[:, 1] + 1.0
    return lax.rsqrt(d)[:, None]


def _tc_first_body(deg_ref, x_ref, w_ref, o0_ref, o1_ref):
    di = _dinv(deg_ref)
    res = jnp.dot(x_ref[...], w_ref[...],
                  preferred_element_type=_F32, precision=_PREC) * di
    h = res.shape[1] // 2
    o0_ref[...] = res[:, :h]
    o1_ref[...] = res[:, h:]


def _tc_layer_body(deg_ref, acc_ref, g0_ref, g1_ref, b_ref, w_ref,
                   o0_ref, o1_ref):
    di = _dinv(deg_ref)
    w = w_ref[...]
    b = b_ref[...]
    hi = w.shape[0] // 2
    t0 = jnp.maximum((acc_ref[0] + g0_ref[...]) * di + b[:, :hi], 0.0)
    t1 = jnp.maximum((acc_ref[1] + g1_ref[...]) * di + b[:, hi:], 0.0)
    res = (jnp.dot(t0, w[:hi], preferred_element_type=_F32, precision=_PREC)
           + jnp.dot(t1, w[hi:], preferred_element_type=_F32, precision=_PREC)
           ) * di
    ho = res.shape[1] // 2
    o0_ref[...] = res[:, :ho]
    o1_ref[...] = res[:, ho:]


def _tc_tail_body(deg_ref, acc_ref, g0_ref, g1_ref, b_ref, o_ref):
    di = _dinv(deg_ref)
    b = b_ref[...]
    hi = b.shape[1] // 2
    t0 = jnp.maximum((acc_ref[0] + g0_ref[...]) * di + b[:, :hi], 0.0)
    t1 = jnp.maximum((acc_ref[1] + g1_ref[...]) * di + b[:, hi:], 0.0)
    t = jnp.concatenate([t0, t1], axis=1)
    m = jnp.max(t, axis=1, keepdims=True)
    lse = jnp.log(jnp.sum(jnp.exp(t - m), axis=1, keepdims=True)) + m
    o_ref[...] = t - lse


def _tc_first(deg_t, x, W, NB):
    N, D_in = x.shape
    D_out = W.shape[1]
    h = D_out // 2
    return pl.pallas_call(
        _tc_first_body,
        grid=(N // NB,),
        in_specs=[
            pl.BlockSpec((NB, 2), lambda i: (i, 0)),
            pl.BlockSpec((NB, D_in), lambda i: (i, 0)),
            pl.BlockSpec((D_in, D_out), lambda i: (0, 0)),
        ],
        out_specs=[pl.BlockSpec((NB, h), lambda i: (i, 0))] * 2,
        out_shape=[jax.ShapeDtypeStruct((N, h), _F32)] * 2,
    )(deg_t, x, W)


def _tc_layer(deg_t, acc, g0, g1, b, W, NB):
    N, hi = g0.shape
    D_out = W.shape[1]
    ho = D_out // 2
    return pl.pallas_call(
        _tc_layer_body,
        grid=(N // NB,),
        in_specs=[
            pl.BlockSpec((NB, 2), lambda i: (i, 0)),
            pl.BlockSpec((2, NB, hi), lambda i: (0, i, 0)),
            pl.BlockSpec((NB, hi), lambda i: (i, 0)),
            pl.BlockSpec((NB, hi), lambda i: (i, 0)),
            pl.BlockSpec((1, 2 * hi), lambda i: (0, 0)),
            pl.BlockSpec((2 * hi, D_out), lambda i: (0, 0)),
        ],
        out_specs=[pl.BlockSpec((NB, ho), lambda i: (i, 0))] * 2,
        out_shape=[jax.ShapeDtypeStruct((N, ho), _F32)] * 2,
    )(deg_t, acc, g0, g1, b, W)


def _tc_tail(deg_t, acc, g0, g1, b, NB):
    N, hi = g0.shape
    return pl.pallas_call(
        _tc_tail_body,
        grid=(N // NB,),
        in_specs=[
            pl.BlockSpec((NB, 2), lambda i: (i, 0)),
            pl.BlockSpec((2, NB, hi), lambda i: (0, i, 0)),
            pl.BlockSpec((NB, hi), lambda i: (i, 0)),
            pl.BlockSpec((NB, hi), lambda i: (i, 0)),
            pl.BlockSpec((1, 2 * hi), lambda i: (0, 0)),
        ],
        out_specs=pl.BlockSpec((NB, 2 * hi), lambda i: (i, 0)),
        out_shape=jax.ShapeDtypeStruct((N, 2 * hi), _F32),
    )(deg_t, acc, g0, g1, b)


def kernel(x, edge_index, W1, b1, W2, b2, W3, b3):
    N, _ = x.shape
    E = edge_index.shape[1]
    D_hid = W2.shape[0]
    D_out = W3.shape[1]
    NB = 2000

    src = edge_index[0]
    dst = edge_index[1]
    ones_k = jnp.ones((1000,), _F32)
    zeros_deg = jnp.zeros((N // 16 - 1,), _F32)
    zeros_h = jnp.zeros((N // 16 - 1, D_hid // 2), _F32)
    zeros_o = jnp.zeros((N // 16 - 1, D_out // 2), _F32)

    degp = _make_deg(N, E)(dst, ones_k, zeros_deg)
    deg_t = degp.reshape(2, N).T

    agg_h = _make_agg(N, E, D_hid // 2)
    agg_o = _make_agg(N, E, D_out // 2)

    g1a, g1b = _tc_first(deg_t, x, W1, NB)
    acc1 = agg_h(g1a, g1b, src, dst, zeros_h).reshape(2, N, D_hid // 2)
    g2a, g2b = _tc_layer(deg_t, acc1, g1a, g1b, b1.reshape(1, -1), W2, NB)
    acc2 = agg_h(g2a, g2b, src, dst, zeros_h).reshape(2, N, D_hid // 2)
    g3a, g3b = _tc_layer(deg_t, acc2, g2a, g2b, b2.reshape(1, -1), W3, NB)
    acc3 = agg_o(g3a, g3b, src, dst, zeros_o).reshape(2, N, D_out // 2)
    return _tc_tail(deg_t, acc3, g3a, g3b, b3.reshape(1, -1), NB)


# speedup vs baseline: 1.0595x; 1.0595x over previous
import functools

import jax
import jax.numpy as jnp
from jax import lax
from jax.experimental import pallas as pl
from jax.experimental.pallas import tpu as pltpu
from jax.experimental.pallas import tpu_sc as plsc

_F32 = jnp.float32


def _make_deg(N, E):
    CH = E // 32
    K = 1000
    assert CH % K == 0 and K % 8 == 0
    nchunks = CH // K
    rows = N // 16 - 1
    assert rows % 8 == 0 and rows * 16 + 16 == N
    mesh = plsc.VectorSubcoreMesh(core_axis_name="c", subcore_axis_name="s")

    @functools.partial(
        pl.kernel, mesh=mesh,
        compiler_params=pltpu.CompilerParams(use_tc_tiling_on_sc=False),
        out_type=jax.ShapeDtypeStruct((2 * N,), _F32),
        scratch_types=[
            pltpu.VMEM((K,), jnp.int32),
            pltpu.VMEM((K,), _F32),
            pltpu.VMEM((rows,), _F32),
            pltpu.VMEM_SHARED((N,), _F32),
        ],
    )
    def deg(dst_hbm, ones_hbm, zeros_hbm, out_hbm, idx_d, ones_v, stage_v, sh_deg):
        c = lax.axis_index("c")
        s = lax.axis_index("s")
        wid = s * 2 + c
        pltpu.sync_copy(ones_hbm, ones_v)
        pltpu.sync_copy(zeros_hbm, stage_v)
        pltpu.sync_copy(stage_v, sh_deg.at[pl.ds(s * rows, rows)])

        @pl.when(s == 15)
        def _():
            pltpu.sync_copy(stage_v.at[pl.ds(0, 16)], sh_deg.at[pl.ds(16 * rows, 16)])

        plsc.subcore_barrier()
        e0 = wid * CH
        for k in range(nchunks):
            pltpu.sync_copy(dst_hbm.at[pl.ds(e0 + k * K, K)], idx_d)
            pltpu.sync_copy(ones_v, sh_deg.at[idx_d], add=True)
        plsc.subcore_barrier()
        pltpu.sync_copy(sh_deg.at[pl.ds(s * rows, rows)], stage_v)
        pltpu.sync_copy(stage_v, out_hbm.at[pl.ds(c * N + s * rows, rows)])

        @pl.when(s == 15)
        def _():
            pltpu.sync_copy(sh_deg.at[pl.ds(16 * rows, 16)], stage_v.at[pl.ds(0, 16)])
            pltpu.sync_copy(stage_v.at[pl.ds(0, 16)],
                            out_hbm.at[pl.ds(c * N + 16 * rows, 16)])

    return deg


def _make_agg(N, E, H):
    CH = E // 16
    K = 400
    assert CH % (2 * K) == 0 and K % 8 == 0
    nchunks = CH // K
    rows = N // 16 - 1
    assert rows % 8 == 0 and rows * 16 + 16 == N
    mesh = plsc.VectorSubcoreMesh(core_axis_name="c", subcore_axis_name="s")

    @functools.partial(
        pl.kernel, mesh=mesh,
        compiler_params=pltpu.CompilerParams(use_tc_tiling_on_sc=False),
        out_type=jax.ShapeDtypeStruct((2 * N, H), _F32),
        scratch_types=[
            pltpu.VMEM((2 * K,), jnp.int32),
            pltpu.VMEM((K,), jnp.int32),
            pltpu.VMEM((K,), jnp.int32),
            pltpu.VMEM((2 * K, H), _F32),
            pltpu.VMEM_SHARED((N, H), _F32),
            pltpu.SemaphoreType.DMA,
            pltpu.SemaphoreType.DMA,
        ],
    )
    def agg(g0_hbm, g1_hbm, src_hbm, dst_hbm, zeros_hbm, out_hbm,
            idx_s, id0, id1, rows_v, sh_acc, sem0, sem1):
        c = lax.axis_index("c")
        s = lax.axis_index("s")
        idb = (id0, id1)
        sems = (sem0, sem1)
        isb = (idx_s.at[pl.ds(0, K)], idx_s.at[pl.ds(K, K)])
        bufs = (rows_v.at[pl.ds(0, K)], rows_v.at[pl.ds(K, K)])
        pltpu.sync_copy(zeros_hbm, rows_v.at[pl.ds(0, rows)])
        pltpu.sync_copy(rows_v.at[pl.ds(0, rows)], sh_acc.at[pl.ds(s * rows, rows)])

        @pl.when(s == 15)
        def _():
            pltpu.sync_copy(rows_v.at[pl.ds(0, 16)], sh_acc.at[pl.ds(16 * rows, 16)])

        plsc.subcore_barrier()
        e0 = s * CH

        def edge_loop(g_hbm):
            handles = [None, None]

            def issue(j, b):
                pltpu.sync_copy(src_hbm.at[pl.ds(e0 + j * K, K)], isb[b])
                pltpu.sync_copy(dst_hbm.at[pl.ds(e0 + j * K, K)], idb[b])
                handles[b] = pltpu.async_copy(g_hbm.at[isb[b]], bufs[b], sems[b])

            issue(0, 0)
            issue(1, 1)
            for j in range(nchunks):
                b = j % 2
                handles[b].wait()
                pltpu.sync_copy(bufs[b], sh_acc.at[idb[b]], add=True)
                if j + 2 < nchunks:
                    issue(j + 2, b)

        @pl.when(c == 0)
        def _():
            edge_loop(g0_hbm)

        @pl.when(c == 1)
        def _():
            edge_loop(g1_hbm)

        plsc.subcore_barrier()
        pltpu.sync_copy(sh_acc.at[pl.ds(s * rows, rows)], rows_v.at[pl.ds(0, rows)])
        pltpu.sync_copy(rows_v.at[pl.ds(0, rows)],
                        out_hbm.at[pl.ds(c * N + s * rows, rows)])

        @pl.when(s == 15)
        def _():
            pltpu.sync_copy(sh_acc.at[pl.ds(16 * rows, 16)], rows_v.at[pl.ds(0, 16)])
            pltpu.sync_copy(rows_v.at[pl.ds(0, 16)],
                            out_hbm.at[pl.ds(c * N + 16 * rows, 16)])

    return agg


_PREC = jax.lax.Precision.HIGHEST


def _dinv(deg_ref):
    d = deg_ref[:, 0] + deg_ref[:, 1] + 1.0
    return lax.rsqrt(d)[:, None]


def _tc_first_body(deg_ref, x_ref, w_ref, o0_ref, o1_ref):
    di = _dinv(deg_ref)
    res = jnp.dot(x_ref[...], w_ref[...],
                  preferred_element_type=_F32, precision=_PREC) * di
    h = res.shape[1] // 2
    o0_ref[...] = res[:, :h]
    o1_ref[...] = res[:, h:]


def _tc_layer_body(deg_ref, acc_ref, g0_ref, g1_ref, b_ref, w_ref,
                   o0_ref, o1_ref):
    di = _dinv(deg_ref)
    w = w_ref[...]
    b = b_ref[...]
    hi = w.shape[0] // 2
    t0 = jnp.maximum((acc_ref[0] + g0_ref[...]) * di + b[:, :hi], 0.0)
    t1 = jnp.maximum((acc_ref[1] + g1_ref[...]) * di + b[:, hi:], 0.0)
    res = (jnp.dot(t0, w[:hi], preferred_element_type=_F32, precision=_PREC)
           + jnp.dot(t1, w[hi:], preferred_element_type=_F32, precision=_PREC)
           ) * di
    ho = res.shape[1] // 2
    o0_ref[...] = res[:, :ho]
    o1_ref[...] = res[:, ho:]


def _tc_tail_body(deg_ref, acc_ref, g0_ref, g1_ref, b_ref, o_ref):
    di = _dinv(deg_ref)
    b = b_ref[...]
    hi = b.shape[1] // 2
    t0 = jnp.maximum((acc_ref[0] + g0_ref[...]) * di + b[:, :hi], 0.0)
    t1 = jnp.maximum((acc_ref[1] + g1_ref[...]) * di + b[:, hi:], 0.0)
    t = jnp.concatenate([t0, t1], axis=1)
    m = jnp.max(t, axis=1, keepdims=True)
    lse = jnp.log(jnp.sum(jnp.exp(t - m), axis=1, keepdims=True)) + m
    o_ref[...] = t - lse


def _tc_first(deg_t, x, W, NB):
    N, D_in = x.shape
    D_out = W.shape[1]
    h = D_out // 2
    return pl.pallas_call(
        _tc_first_body,
        grid=(N // NB,),
        in_specs=[
            pl.BlockSpec((NB, 2), lambda i: (i, 0)),
            pl.BlockSpec((NB, D_in), lambda i: (i, 0)),
            pl.BlockSpec((D_in, D_out), lambda i: (0, 0)),
        ],
        out_specs=[pl.BlockSpec((NB, h), lambda i: (i, 0))] * 2,
        out_shape=[jax.ShapeDtypeStruct((N, h), _F32)] * 2,
    )(deg_t, x, W)


def _tc_layer(deg_t, acc, g0, g1, b, W, NB):
    N, hi = g0.shape
    D_out = W.shape[1]
    ho = D_out // 2
    return pl.pallas_call(
        _tc_layer_body,
        grid=(N // NB,),
        in_specs=[
            pl.BlockSpec((NB, 2), lambda i: (i, 0)),
            pl.BlockSpec((2, NB, hi), lambda i: (0, i, 0)),
            pl.BlockSpec((NB, hi), lambda i: (i, 0)),
            pl.BlockSpec((NB, hi), lambda i: (i, 0)),
            pl.BlockSpec((1, 2 * hi), lambda i: (0, 0)),
            pl.BlockSpec((2 * hi, D_out), lambda i: (0, 0)),
        ],
        out_specs=[pl.BlockSpec((NB, ho), lambda i: (i, 0))] * 2,
        out_shape=[jax.ShapeDtypeStruct((N, ho), _F32)] * 2,
    )(deg_t, acc, g0, g1, b, W)


def _tc_tail(deg_t, acc, g0, g1, b, NB):
    N, hi = g0.shape
    return pl.pallas_call(
        _tc_tail_body,
        grid=(N // NB,),
        in_specs=[
            pl.BlockSpec((NB, 2), lambda i: (i, 0)),
            pl.BlockSpec((2, NB, hi), lambda i: (0, i, 0)),
            pl.BlockSpec((NB, hi), lambda i: (i, 0)),
            pl.BlockSpec((NB, hi), lambda i: (i, 0)),
            pl.BlockSpec((1, 2 * hi), lambda i: (0, 0)),
        ],
        out_specs=pl.BlockSpec((NB, 2 * hi), lambda i: (i, 0)),
        out_shape=jax.ShapeDtypeStruct((N, 2 * hi), _F32),
    )(deg_t, acc, g0, g1, b)


def kernel(x, edge_index, W1, b1, W2, b2, W3, b3):
    N, _ = x.shape
    E = edge_index.shape[1]
    D_hid = W2.shape[0]
    D_out = W3.shape[1]
    NB = 2000

    src = edge_index[0]
    dst = edge_index[1]
    ones_k = jnp.ones((1000,), _F32)
    zeros_deg = jnp.zeros((N // 16 - 1,), _F32)
    zeros_h = jnp.zeros((N // 16 - 1, D_hid // 2), _F32)
    zeros_o = jnp.zeros((N // 16 - 1, D_out // 2), _F32)

    degp = _make_deg(N, E)(dst, ones_k, zeros_deg)
    deg_t = degp.reshape(2, N).T

    agg_h = _make_agg(N, E, D_hid // 2)
    agg_o = _make_agg(N, E, D_out // 2)

    g1a, g1b = _tc_first(deg_t, x, W1, NB)
    acc1 = agg_h(g1a, g1b, src, dst, zeros_h).reshape(2, N, D_hid // 2)
    g2a, g2b = _tc_layer(deg_t, acc1, g1a, g1b, b1.reshape(1, -1), W2, NB)
    acc2 = agg_h(g2a, g2b, src, dst, zeros_h).reshape(2, N, D_hid // 2)
    g3a, g3b = _tc_layer(deg_t, acc2, g2a, g2b, b2.reshape(1, -1), W3, NB)
    acc3 = agg_o(g3a, g3b, src, dst, zeros_o).reshape(2, N, D_out // 2)
    return _tc_tail(deg_t, acc3, g3a, g3b, b3.reshape(1, -1), NB)


# speedup vs baseline: 1.1071x; 1.0449x over previous
import functools

import jax
import jax.numpy as jnp
from jax import lax
from jax.experimental import pallas as pl
from jax.experimental.pallas import tpu as pltpu
from jax.experimental.pallas import tpu_sc as plsc

_F32 = jnp.float32


def _make_deg(N, E):
    CH = E // 32
    K = 1000
    assert CH % K == 0 and K % 8 == 0
    nchunks = CH // K
    rows = N // 16 - 1
    assert rows % 8 == 0 and rows * 16 + 16 == N
    mesh = plsc.VectorSubcoreMesh(core_axis_name="c", subcore_axis_name="s")

    @functools.partial(
        pl.kernel, mesh=mesh,
        compiler_params=pltpu.CompilerParams(use_tc_tiling_on_sc=False),
        out_type=jax.ShapeDtypeStruct((2 * N,), _F32),
        scratch_types=[
            pltpu.VMEM((K,), jnp.int32),
            pltpu.VMEM((K,), _F32),
            pltpu.VMEM((rows,), _F32),
            pltpu.VMEM_SHARED((N,), _F32),
        ],
    )
    def deg(dst_hbm, ones_hbm, zeros_hbm, out_hbm, idx_d, ones_v, stage_v, sh_deg):
        c = lax.axis_index("c")
        s = lax.axis_index("s")
        wid = s * 2 + c
        pltpu.sync_copy(ones_hbm, ones_v)
        pltpu.sync_copy(zeros_hbm, stage_v)
        pltpu.sync_copy(stage_v, sh_deg.at[pl.ds(s * rows, rows)])

        @pl.when(s == 15)
        def _():
            pltpu.sync_copy(stage_v.at[pl.ds(0, 16)], sh_deg.at[pl.ds(16 * rows, 16)])

        plsc.subcore_barrier()
        e0 = wid * CH
        for k in range(nchunks):
            pltpu.sync_copy(dst_hbm.at[pl.ds(e0 + k * K, K)], idx_d)
            pltpu.sync_copy(ones_v, sh_deg.at[idx_d], add=True)
        plsc.subcore_barrier()
        pltpu.sync_copy(sh_deg.at[pl.ds(s * rows, rows)], stage_v)
        pltpu.sync_copy(stage_v, out_hbm.at[pl.ds(c * N + s * rows, rows)])

        @pl.when(s == 15)
        def _():
            pltpu.sync_copy(sh_deg.at[pl.ds(16 * rows, 16)], stage_v.at[pl.ds(0, 16)])
            pltpu.sync_copy(stage_v.at[pl.ds(0, 16)],
                            out_hbm.at[pl.ds(c * N + 16 * rows, 16)])

    return deg


def _make_agg(N, E, H):
    CH = E // 16
    K = 400
    assert CH % (2 * K) == 0 and K % 8 == 0
    nchunks = CH // K
    rows = N // 16 - 1
    assert rows % 8 == 0 and rows * 16 + 16 == N
    mesh = plsc.VectorSubcoreMesh(core_axis_name="c", subcore_axis_name="s")

    @functools.partial(
        pl.kernel, mesh=mesh,
        compiler_params=pltpu.CompilerParams(use_tc_tiling_on_sc=False),
        out_type=jax.ShapeDtypeStruct((2 * N, H), _F32),
        scratch_types=[
            pltpu.VMEM((2 * K,), jnp.int32),
            pltpu.VMEM((K,), jnp.int32),
            pltpu.VMEM((K,), jnp.int32),
            pltpu.VMEM((2 * K, H), _F32),
            pltpu.VMEM_SHARED((N, H), _F32),
            pltpu.SemaphoreType.DMA,
            pltpu.SemaphoreType.DMA,
        ],
    )
    def agg(g0_hbm, g1_hbm, src_hbm, dst_hbm, zeros_hbm, out_hbm,
            idx_s, id0, id1, rows_v, sh_acc, sem0, sem1):
        c = lax.axis_index("c")
        s = lax.axis_index("s")
        idb = (id0, id1)
        sems = (sem0, sem1)
        isb = (idx_s.at[pl.ds(0, K)], idx_s.at[pl.ds(K, K)])
        bufs = (rows_v.at[pl.ds(0, K)], rows_v.at[pl.ds(K, K)])
        pltpu.sync_copy(zeros_hbm, rows_v.at[pl.ds(0, rows)])
        pltpu.sync_copy(rows_v.at[pl.ds(0, rows)], sh_acc.at[pl.ds(s * rows, rows)])

        @pl.when(s == 15)
        def _():
            pltpu.sync_copy(rows_v.at[pl.ds(0, 16)], sh_acc.at[pl.ds(16 * rows, 16)])

        plsc.subcore_barrier()
        e0 = s * CH

        def edge_loop(g_hbm):
            handles = [None, None]

            def issue(j, b):
                pltpu.sync_copy(src_hbm.at[pl.ds(e0 + j * K, K)], isb[b])
                pltpu.sync_copy(dst_hbm.at[pl.ds(e0 + j * K, K)], idb[b])
                handles[b] = pltpu.async_copy(g_hbm.at[isb[b]], bufs[b], sems[b])

            issue(0, 0)
            issue(1, 1)
            for j in range(nchunks):
                b = j % 2
                handles[b].wait()
                pltpu.sync_copy(bufs[b], sh_acc.at[idb[b]], add=True)
                if j + 2 < nchunks:
                    issue(j + 2, b)

        @pl.when(c == 0)
        def _():
            edge_loop(g0_hbm)

        @pl.when(c == 1)
        def _():
            edge_loop(g1_hbm)

        plsc.subcore_barrier()
        pltpu.sync_copy(sh_acc.at[pl.ds(s * rows, rows)], rows_v.at[pl.ds(0, rows)])
        pltpu.sync_copy(rows_v.at[pl.ds(0, rows)],
                        out_hbm.at[pl.ds(c * N + s * rows, rows)])

        @pl.when(s == 15)
        def _():
            pltpu.sync_copy(sh_acc.at[pl.ds(16 * rows, 16)], rows_v.at[pl.ds(0, 16)])
            pltpu.sync_copy(rows_v.at[pl.ds(0, 16)],
                            out_hbm.at[pl.ds(c * N + 16 * rows, 16)])

    return agg


def _make_agg_part(N, E, D):
    CH = E // 32
    K = 400
    assert CH % K == 0 and K % 8 == 0
    nchunks = CH // K
    rows = N // 16 - 1
    assert rows % 8 == 0 and rows * 16 + 16 == N
    mesh = plsc.VectorSubcoreMesh(core_axis_name="c", subcore_axis_name="s")

    @functools.partial(
        pl.kernel, mesh=mesh,
        compiler_params=pltpu.CompilerParams(use_tc_tiling_on_sc=False),
        out_type=jax.ShapeDtypeStruct((2 * N, D), _F32),
        scratch_types=[
            pltpu.VMEM((2 * K,), jnp.int32),
            pltpu.VMEM((K,), jnp.int32),
            pltpu.VMEM((K,), jnp.int32),
            pltpu.VMEM((2 * K, D), _F32),
            pltpu.VMEM_SHARED((N, D), _F32),
            pltpu.SemaphoreType.DMA,
            pltpu.SemaphoreType.DMA,
        ],
    )
    def agg(g_hbm, src_hbm, dst_hbm, zeros_hbm, out_hbm,
            idx_s, id0, id1, rows_v, sh_acc, sem0, sem1):
        c = lax.axis_index("c")
        s = lax.axis_index("s")
        idb = (id0, id1)
        sems = (sem0, sem1)
        isb = (idx_s.at[pl.ds(0, K)], idx_s.at[pl.ds(K, K)])
        bufs = (rows_v.at[pl.ds(0, K)], rows_v.at[pl.ds(K, K)])
        pltpu.sync_copy(zeros_hbm, rows_v.at[pl.ds(0, rows)])
        pltpu.sync_copy(rows_v.at[pl.ds(0, rows)], sh_acc.at[pl.ds(s * rows, rows)])

        @pl.when(s == 15)
        def _():
            pltpu.sync_copy(rows_v.at[pl.ds(0, 16)], sh_acc.at[pl.ds(16 * rows, 16)])

        plsc.subcore_barrier()
        wid = s * 2 + c
        e0 = wid * CH
        handles = [None, None]

        def issue(j, b):
            pltpu.sync_copy(src_hbm.at[pl.ds(e0 + j * K, K)], isb[b])
            pltpu.sync_copy(dst_hbm.at[pl.ds(e0 + j * K, K)], idb[b])
            handles[b] = pltpu.async_copy(g_hbm.at[isb[b]], bufs[b], sems[b])

        issue(0, 0)
        issue(1, 1)
        for j in range(nchunks):
            b = j % 2
            handles[b].wait()
            pltpu.sync_copy(bufs[b], sh_acc.at[idb[b]], add=True)
            if j + 2 < nchunks:
                issue(j + 2, b)
        plsc.subcore_barrier()
        pltpu.sync_copy(sh_acc.at[pl.ds(s * rows, rows)], rows_v.at[pl.ds(0, rows)])
        pltpu.sync_copy(rows_v.at[pl.ds(0, rows)],
                        out_hbm.at[pl.ds(c * N + s * rows, rows)])

        @pl.when(s == 15)
        def _():
            pltpu.sync_copy(sh_acc.at[pl.ds(16 * rows, 16)], rows_v.at[pl.ds(0, 16)])
            pltpu.sync_copy(rows_v.at[pl.ds(0, 16)],
                            out_hbm.at[pl.ds(c * N + 16 * rows, 16)])

    return agg


_PREC = jax.lax.Precision.HIGHEST


def _dinv(deg_ref):
    d = deg_ref[:, 0] + deg_ref[:, 1] + 1.0
    return lax.rsqrt(d)[:, None]


def _tc_first_body(deg_ref, x_ref, w_ref, o0_ref, o1_ref):
    di = _dinv(deg_ref)
    res = jnp.dot(x_ref[...], w_ref[...],
                  preferred_element_type=_F32, precision=_PREC) * di
    h = res.shape[1] // 2
    o0_ref[...] = res[:, :h]
    o1_ref[...] = res[:, h:]


def _tc_layer_body(deg_ref, acc_ref, g0_ref, g1_ref, b_ref, w_ref,
                   o0_ref, o1_ref):
    di = _dinv(deg_ref)
    w = w_ref[...]
    b = b_ref[...]
    hi = w.shape[0] // 2
    t0 = jnp.maximum((acc_ref[0] + g0_ref[...]) * di + b[:, :hi], 0.0)
    t1 = jnp.maximum((acc_ref[1] + g1_ref[...]) * di + b[:, hi:], 0.0)
    res = (jnp.dot(t0, w[:hi], preferred_element_type=_F32, precision=_PREC)
           + jnp.dot(t1, w[hi:], preferred_element_type=_F32, precision=_PREC)
           ) * di
    ho = res.shape[1] // 2
    o0_ref[...] = res[:, :ho]
    o1_ref[...] = res[:, ho:]


def _tc_layer1_body(deg_ref, acc_ref, g0_ref, g1_ref, b_ref, w_ref, o_ref):
    di = _dinv(deg_ref)
    w = w_ref[...]
    b = b_ref[...]
    hi = w.shape[0] // 2
    t0 = jnp.maximum((acc_ref[0] + g0_ref[...]) * di + b[:, :hi], 0.0)
    t1 = jnp.maximum((acc_ref[1] + g1_ref[...]) * di + b[:, hi:], 0.0)
    o_ref[...] = (jnp.dot(t0, w[:hi], preferred_element_type=_F32,
                          precision=_PREC)
                  + jnp.dot(t1, w[hi:], preferred_element_type=_F32,
                            precision=_PREC)) * di


def _tc_tail_part_body(deg_ref, acc_ref, g_ref, b_ref, o_ref):
    di = _dinv(deg_ref)
    t = (acc_ref[0] + acc_ref[1] + g_ref[...]) * di + b_ref[...]
    t = jnp.maximum(t, 0.0)
    m = jnp.max(t, axis=1, keepdims=True)
    lse = jnp.log(jnp.sum(jnp.exp(t - m), axis=1, keepdims=True)) + m
    o_ref[...] = t - lse


def _tc_tail_body(deg_ref, acc_ref, g0_ref, g1_ref, b_ref, o_ref):
    di = _dinv(deg_ref)
    b = b_ref[...]
    hi = b.shape[1] // 2
    t0 = jnp.maximum((acc_ref[0] + g0_ref[...]) * di + b[:, :hi], 0.0)
    t1 = jnp.maximum((acc_ref[1] + g1_ref[...]) * di + b[:, hi:], 0.0)
    t = jnp.concatenate([t0, t1], axis=1)
    m = jnp.max(t, axis=1, keepdims=True)
    lse = jnp.log(jnp.sum(jnp.exp(t - m), axis=1, keepdims=True)) + m
    o_ref[...] = t - lse


def _tc_first(deg_t, x, W, NB):
    N, D_in = x.shape
    D_out = W.shape[1]
    h = D_out // 2
    return pl.pallas_call(
        _tc_first_body,
        grid=(N // NB,),
        in_specs=[
            pl.BlockSpec((NB, 2), lambda i: (i, 0)),
            pl.BlockSpec((NB, D_in), lambda i: (i, 0)),
            pl.BlockSpec((D_in, D_out), lambda i: (0, 0)),
        ],
        out_specs=[pl.BlockSpec((NB, h), lambda i: (i, 0))] * 2,
        out_shape=[jax.ShapeDtypeStruct((N, h), _F32)] * 2,
    )(deg_t, x, W)


def _tc_layer(deg_t, acc, g0, g1, b, W, NB):
    N, hi = g0.shape
    D_out = W.shape[1]
    ho = D_out // 2
    return pl.pallas_call(
        _tc_layer_body,
        grid=(N // NB,),
        in_specs=[
            pl.BlockSpec((NB, 2), lambda i: (i, 0)),
            pl.BlockSpec((2, NB, hi), lambda i: (0, i, 0)),
            pl.BlockSpec((NB, hi), lambda i: (i, 0)),
            pl.BlockSpec((NB, hi), lambda i: (i, 0)),
            pl.BlockSpec((1, 2 * hi), lambda i: (0, 0)),
            pl.BlockSpec((2 * hi, D_out), lambda i: (0, 0)),
        ],
        out_specs=[pl.BlockSpec((NB, ho), lambda i: (i, 0))] * 2,
        out_shape=[jax.ShapeDtypeStruct((N, ho), _F32)] * 2,
    )(deg_t, acc, g0, g1, b, W)


def _tc_layer1(deg_t, acc, g0, g1, b, W, NB):
    N, hi = g0.shape
    D_out = W.shape[1]
    return pl.pallas_call(
        _tc_layer1_body,
        grid=(N // NB,),
        in_specs=[
            pl.BlockSpec((NB, 2), lambda i: (i, 0)),
            pl.BlockSpec((2, NB, hi), lambda i: (0, i, 0)),
            pl.BlockSpec((NB, hi), lambda i: (i, 0)),
            pl.BlockSpec((NB, hi), lambda i: (i, 0)),
            pl.BlockSpec((1, 2 * hi), lambda i: (0, 0)),
            pl.BlockSpec((2 * hi, D_out), lambda i: (0, 0)),
        ],
        out_specs=pl.BlockSpec((NB, D_out), lambda i: (i, 0)),
        out_shape=jax.ShapeDtypeStruct((N, D_out), _F32),
    )(deg_t, acc, g0, g1, b, W)


def _tc_tail_part(deg_t, acc, g, b, NB):
    N, D = g.shape
    return pl.pallas_call(
        _tc_tail_part_body,
        grid=(N // NB,),
        in_specs=[
            pl.BlockSpec((NB, 2), lambda i: (i, 0)),
            pl.BlockSpec((2, NB, D), lambda i: (0, i, 0)),
            pl.BlockSpec((NB, D), lambda i: (i, 0)),
            pl.BlockSpec((1, D), lambda i: (0, 0)),
        ],
        out_specs=pl.BlockSpec((NB, D), lambda i: (i, 0)),
        out_shape=jax.ShapeDtypeStruct((N, D), _F32),
    )(deg_t, acc, g, b)


def _tc_tail(deg_t, acc, g0, g1, b, NB):
    N, hi = g0.shape
    return pl.pallas_call(
        _tc_tail_body,
        grid=(N // NB,),
        in_specs=[
            pl.BlockSpec((NB, 2), lambda i: (i, 0)),
            pl.BlockSpec((2, NB, hi), lambda i: (0, i, 0)),
            pl.BlockSpec((NB, hi), lambda i: (i, 0)),
            pl.BlockSpec((NB, hi), lambda i: (i, 0)),
            pl.BlockSpec((1, 2 * hi), lambda i: (0, 0)),
        ],
        out_specs=pl.BlockSpec((NB, 2 * hi), lambda i: (i, 0)),
        out_shape=jax.ShapeDtypeStruct((N, 2 * hi), _F32),
    )(deg_t, acc, g0, g1, b)


def kernel(x, edge_index, W1, b1, W2, b2, W3, b3):
    N, _ = x.shape
    E = edge_index.shape[1]
    D_hid = W2.shape[0]
    D_out = W3.shape[1]
    NB = 2000

    src = edge_index[0]
    dst = edge_index[1]
    ones_k = jnp.ones((1000,), _F32)
    zeros_deg = jnp.zeros((N // 16 - 1,), _F32)
    zeros_h = jnp.zeros((N // 16 - 1, D_hid // 2), _F32)
    zeros_o = jnp.zeros((N // 16 - 1, D_out), _F32)

    degp = _make_deg(N, E)(dst, ones_k, zeros_deg)
    deg_t = degp.reshape(2, N).T

    agg_h = _make_agg(N, E, D_hid // 2)
    agg_o = _make_agg_part(N, E, D_out)

    g1a, g1b = _tc_first(deg_t, x, W1, NB)
    acc1 = agg_h(g1a, g1b, src, dst, zeros_h).reshape(2, N, D_hid // 2)
    g2a, g2b = _tc_layer(deg_t, acc1, g1a, g1b, b1.reshape(1, -1), W2, NB)
    acc2 = agg_h(g2a, g2b, src, dst, zeros_h).reshape(2, N, D_hid // 2)
    g3 = _tc_layer1(deg_t, acc2, g2a, g2b, b2.reshape(1, -1), W3, NB)
    acc3 = agg_o(g3, src, dst, zeros_o).reshape(2, N, D_out)
    return _tc_tail_part(deg_t, acc3, g3, b3.reshape(1, -1), NB)


# speedup vs baseline: 1.2220x; 1.1038x over previous
import functools

import jax
import jax.numpy as jnp
from jax import lax
from jax.experimental import pallas as pl
from jax.experimental.pallas import tpu as pltpu
from jax.experimental.pallas import tpu_sc as plsc

_F32 = jnp.float32


def _make_deg(N, E):
    CH = E // 32
    K = 1000
    assert CH % K == 0 and K % 8 == 0
    nchunks = CH // K
    rows = N // 16 - 1
    assert rows % 8 == 0 and rows * 16 + 16 == N
    mesh = plsc.VectorSubcoreMesh(core_axis_name="c", subcore_axis_name="s")

    @functools.partial(
        pl.kernel, mesh=mesh,
        compiler_params=pltpu.CompilerParams(use_tc_tiling_on_sc=False),
        out_type=jax.ShapeDtypeStruct((2 * N,), _F32),
        scratch_types=[
            pltpu.VMEM((K,), jnp.int32),
            pltpu.VMEM((K,), _F32),
            pltpu.VMEM((rows,), _F32),
            pltpu.VMEM_SHARED((N,), _F32),
        ],
    )
    def deg(dst_hbm, ones_hbm, zeros_hbm, out_hbm, idx_d, ones_v, stage_v, sh_deg):
        c = lax.axis_index("c")
        s = lax.axis_index("s")
        wid = s * 2 + c
        pltpu.sync_copy(ones_hbm, ones_v)
        pltpu.sync_copy(zeros_hbm, stage_v)
        pltpu.sync_copy(stage_v, sh_deg.at[pl.ds(s * rows, rows)])

        @pl.when(s == 15)
        def _():
            pltpu.sync_copy(stage_v.at[pl.ds(0, 16)], sh_deg.at[pl.ds(16 * rows, 16)])

        plsc.subcore_barrier()
        e0 = wid * CH
        for k in range(nchunks):
            pltpu.sync_copy(dst_hbm.at[pl.ds(e0 + k * K, K)], idx_d)
            pltpu.sync_copy(ones_v, sh_deg.at[idx_d], add=True)
        plsc.subcore_barrier()
        pltpu.sync_copy(sh_deg.at[pl.ds(s * rows, rows)], stage_v)
        pltpu.sync_copy(stage_v, out_hbm.at[pl.ds(c * N + s * rows, rows)])

        @pl.when(s == 15)
        def _():
            pltpu.sync_copy(sh_deg.at[pl.ds(16 * rows, 16)], stage_v.at[pl.ds(0, 16)])
            pltpu.sync_copy(stage_v.at[pl.ds(0, 16)],
                            out_hbm.at[pl.ds(c * N + 16 * rows, 16)])

    return deg


def _make_agg(N, E, H):
    CH = E // 16
    K = 400
    assert CH % (2 * K) == 0 and K % 8 == 0
    nchunks = CH // K
    rows = N // 16 - 1
    assert rows % 8 == 0 and rows * 16 + 16 == N
    mesh = plsc.VectorSubcoreMesh(core_axis_name="c", subcore_axis_name="s")

    @functools.partial(
        pl.kernel, mesh=mesh,
        compiler_params=pltpu.CompilerParams(use_tc_tiling_on_sc=False),
        out_type=jax.ShapeDtypeStruct((2 * N, H), _F32),
        scratch_types=[
            pltpu.VMEM((3 * K,), jnp.int32),
            [pltpu.VMEM((K,), jnp.int32)] * 3,
            pltpu.VMEM((3 * K, H), _F32),
            pltpu.VMEM_SHARED((N, H), _F32),
            [pltpu.SemaphoreType.DMA] * 3,
            [pltpu.SemaphoreType.DMA] * 3,
        ],
    )
    def agg(g0_hbm, g1_hbm, src_hbm, dst_hbm, zeros_hbm, out_hbm,
            idx_s, idb, rows_v, sh_acc, gsems, ssems):
        c = lax.axis_index("c")
        s = lax.axis_index("s")
        isb = [idx_s.at[pl.ds(i * K, K)] for i in range(3)]
        bufs = [rows_v.at[pl.ds(i * K, K)] for i in range(3)]
        pltpu.sync_copy(zeros_hbm, rows_v.at[pl.ds(0, rows)])
        pltpu.sync_copy(rows_v.at[pl.ds(0, rows)], sh_acc.at[pl.ds(s * rows, rows)])

        @pl.when(s == 15)
        def _():
            pltpu.sync_copy(rows_v.at[pl.ds(0, 16)], sh_acc.at[pl.ds(16 * rows, 16)])

        plsc.subcore_barrier()
        e0 = s * CH

        def edge_loop(g_hbm):
            gh = [None] * 3
            sh = [None] * 3

            def issue(j, b):
                pltpu.sync_copy(src_hbm.at[pl.ds(e0 + j * K, K)], isb[b])
                pltpu.sync_copy(dst_hbm.at[pl.ds(e0 + j * K, K)], idb[b])
                gh[b] = pltpu.async_copy(g_hbm.at[isb[b]], bufs[b], gsems[b])

            issue(0, 0)
            issue(1, 1)
            for j in range(nchunks):
                b = j % 3
                gh[b].wait()
                sh[b] = pltpu.async_copy(bufs[b], sh_acc.at[idb[b]], ssems[b],
                                         add=True)
                if j + 2 < nchunks:
                    b2 = (j + 2) % 3
                    if sh[b2] is not None:
                        sh[b2].wait()
                        sh[b2] = None
                    issue(j + 2, b2)
            for b in range(3):
                if sh[b] is not None:
                    sh[b].wait()

        @pl.when(c == 0)
        def _():
            edge_loop(g0_hbm)

        @pl.when(c == 1)
        def _():
            edge_loop(g1_hbm)

        plsc.subcore_barrier()
        pltpu.sync_copy(sh_acc.at[pl.ds(s * rows, rows)], rows_v.at[pl.ds(0, rows)])
        pltpu.sync_copy(rows_v.at[pl.ds(0, rows)],
                        out_hbm.at[pl.ds(c * N + s * rows, rows)])

        @pl.when(s == 15)
        def _():
            pltpu.sync_copy(sh_acc.at[pl.ds(16 * rows, 16)], rows_v.at[pl.ds(0, 16)])
            pltpu.sync_copy(rows_v.at[pl.ds(0, 16)],
                            out_hbm.at[pl.ds(c * N + 16 * rows, 16)])

    return agg


def _make_agg_part(N, E, D):
    CH = E // 32
    K = 400
    assert CH % K == 0 and K % 8 == 0
    nchunks = CH // K
    rows = N // 16 - 1
    assert rows % 8 == 0 and rows * 16 + 16 == N
    mesh = plsc.VectorSubcoreMesh(core_axis_name="c", subcore_axis_name="s")

    @functools.partial(
        pl.kernel, mesh=mesh,
        compiler_params=pltpu.CompilerParams(use_tc_tiling_on_sc=False),
        out_type=jax.ShapeDtypeStruct((2 * N, D), _F32),
        scratch_types=[
            pltpu.VMEM((3 * K,), jnp.int32),
            [pltpu.VMEM((K,), jnp.int32)] * 3,
            pltpu.VMEM((3 * K, D), _F32),
            pltpu.VMEM_SHARED((N, D), _F32),
            [pltpu.SemaphoreType.DMA] * 3,
            [pltpu.SemaphoreType.DMA] * 3,
        ],
    )
    def agg(g_hbm, src_hbm, dst_hbm, zeros_hbm, out_hbm,
            idx_s, idb, rows_v, sh_acc, gsems, ssems):
        c = lax.axis_index("c")
        s = lax.axis_index("s")
        isb = [idx_s.at[pl.ds(i * K, K)] for i in range(3)]
        bufs = [rows_v.at[pl.ds(i * K, K)] for i in range(3)]
        pltpu.sync_copy(zeros_hbm, rows_v.at[pl.ds(0, rows)])
        pltpu.sync_copy(rows_v.at[pl.ds(0, rows)], sh_acc.at[pl.ds(s * rows, rows)])

        @pl.when(s == 15)
        def _():
            pltpu.sync_copy(rows_v.at[pl.ds(0, 16)], sh_acc.at[pl.ds(16 * rows, 16)])

        plsc.subcore_barrier()
        wid = s * 2 + c
        e0 = wid * CH
        gh = [None] * 3
        sh = [None] * 3

        def issue(j, b):
            pltpu.sync_copy(src_hbm.at[pl.ds(e0 + j * K, K)], isb[b])
            pltpu.sync_copy(dst_hbm.at[pl.ds(e0 + j * K, K)], idb[b])
            gh[b] = pltpu.async_copy(g_hbm.at[isb[b]], bufs[b], gsems[b])

        issue(0, 0)
        issue(1, 1)
        for j in range(nchunks):
            b = j % 3
            gh[b].wait()
            sh[b] = pltpu.async_copy(bufs[b], sh_acc.at[idb[b]], ssems[b],
                                     add=True)
            if j + 2 < nchunks:
                b2 = (j + 2) % 3
                if sh[b2] is not None:
                    sh[b2].wait()
                    sh[b2] = None
                issue(j + 2, b2)
        for b in range(3):
            if sh[b] is not None:
                sh[b].wait()
        plsc.subcore_barrier()
        pltpu.sync_copy(sh_acc.at[pl.ds(s * rows, rows)], rows_v.at[pl.ds(0, rows)])
        pltpu.sync_copy(rows_v.at[pl.ds(0, rows)],
                        out_hbm.at[pl.ds(c * N + s * rows, rows)])

        @pl.when(s == 15)
        def _():
            pltpu.sync_copy(sh_acc.at[pl.ds(16 * rows, 16)], rows_v.at[pl.ds(0, 16)])
            pltpu.sync_copy(rows_v.at[pl.ds(0, 16)],
                            out_hbm.at[pl.ds(c * N + 16 * rows, 16)])

    return agg


_PREC = jax.lax.Precision.HIGHEST


def _dinv(deg_ref):
    d = deg_ref[:, 0] + deg_ref[:, 1] + 1.0
    return lax.rsqrt(d)[:, None]


def _tc_first_body(deg_ref, x_ref, w_ref, o0_ref, o1_ref):
    di = _dinv(deg_ref)
    res = jnp.dot(x_ref[...], w_ref[...],
                  preferred_element_type=_F32, precision=_PREC) * di
    h = res.shape[1] // 2
    o0_ref[...] = res[:, :h]
    o1_ref[...] = res[:, h:]


def _tc_layer_body(deg_ref, acc_ref, g0_ref, g1_ref, b_ref, w_ref,
                   o0_ref, o1_ref):
    di = _dinv(deg_ref)
    w = w_ref[...]
    b = b_ref[...]
    hi = w.shape[0] // 2
    t0 = jnp.maximum((acc_ref[0] + g0_ref[...]) * di + b[:, :hi], 0.0)
    t1 = jnp.maximum((acc_ref[1] + g1_ref[...]) * di + b[:, hi:], 0.0)
    res = (jnp.dot(t0, w[:hi], preferred_element_type=_F32, precision=_PREC)
           + jnp.dot(t1, w[hi:], preferred_element_type=_F32, precision=_PREC)
           ) * di
    ho = res.shape[1] // 2
    o0_ref[...] = res[:, :ho]
    o1_ref[...] = res[:, ho:]


def _tc_layer1_body(deg_ref, acc_ref, g0_ref, g1_ref, b_ref, w_ref, o_ref):
    di = _dinv(deg_ref)
    w = w_ref[...]
    b = b_ref[...]
    hi = w.shape[0] // 2
    t0 = jnp.maximum((acc_ref[0] + g0_ref[...]) * di + b[:, :hi], 0.0)
    t1 = jnp.maximum((acc_ref[1] + g1_ref[...]) * di + b[:, hi:], 0.0)
    o_ref[...] = (jnp.dot(t0, w[:hi], preferred_element_type=_F32,
                          precision=_PREC)
                  + jnp.dot(t1, w[hi:], preferred_element_type=_F32,
                            precision=_PREC)) * di


def _tc_tail_part_body(deg_ref, acc_ref, g_ref, b_ref, o_ref):
    di = _dinv(deg_ref)
    t = (acc_ref[0] + acc_ref[1] + g_ref[...]) * di + b_ref[...]
    t = jnp.maximum(t, 0.0)
    m = jnp.max(t, axis=1, keepdims=True)
    lse = jnp.log(jnp.sum(jnp.exp(t - m), axis=1, keepdims=True)) + m
    o_ref[...] = t - lse


def _tc_tail_body(deg_ref, acc_ref, g0_ref, g1_ref, b_ref, o_ref):
    di = _dinv(deg_ref)
    b = b_ref[...]
    hi = b.shape[1] // 2
    t0 = jnp.maximum((acc_ref[0] + g0_ref[...]) * di + b[:, :hi], 0.0)
    t1 = jnp.maximum((acc_ref[1] + g1_ref[...]) * di + b[:, hi:], 0.0)
    t = jnp.concatenate([t0, t1], axis=1)
    m = jnp.max(t, axis=1, keepdims=True)
    lse = jnp.log(jnp.sum(jnp.exp(t - m), axis=1, keepdims=True)) + m
    o_ref[...] = t - lse


def _tc_first(deg_t, x, W, NB):
    N, D_in = x.shape
    D_out = W.shape[1]
    h = D_out // 2
    return pl.pallas_call(
        _tc_first_body,
        grid=(N // NB,),
        in_specs=[
            pl.BlockSpec((NB, 2), lambda i: (i, 0)),
            pl.BlockSpec((NB, D_in), lambda i: (i, 0)),
            pl.BlockSpec((D_in, D_out), lambda i: (0, 0)),
        ],
        out_specs=[pl.BlockSpec((NB, h), lambda i: (i, 0))] * 2,
        out_shape=[jax.ShapeDtypeStruct((N, h), _F32)] * 2,
    )(deg_t, x, W)


def _tc_layer(deg_t, acc, g0, g1, b, W, NB):
    N, hi = g0.shape
    D_out = W.shape[1]
    ho = D_out // 2
    return pl.pallas_call(
        _tc_layer_body,
        grid=(N // NB,),
        in_specs=[
            pl.BlockSpec((NB, 2), lambda i: (i, 0)),
            pl.BlockSpec((2, NB, hi), lambda i: (0, i, 0)),
            pl.BlockSpec((NB, hi), lambda i: (i, 0)),
            pl.BlockSpec((NB, hi), lambda i: (i, 0)),
            pl.BlockSpec((1, 2 * hi), lambda i: (0, 0)),
            pl.BlockSpec((2 * hi, D_out), lambda i: (0, 0)),
        ],
        out_specs=[pl.BlockSpec((NB, ho), lambda i: (i, 0))] * 2,
        out_shape=[jax.ShapeDtypeStruct((N, ho), _F32)] * 2,
    )(deg_t, acc, g0, g1, b, W)


def _tc_layer1(deg_t, acc, g0, g1, b, W, NB):
    N, hi = g0.shape
    D_out = W.shape[1]
    return pl.pallas_call(
        _tc_layer1_body,
        grid=(N // NB,),
        in_specs=[
            pl.BlockSpec((NB, 2), lambda i: (i, 0)),
            pl.BlockSpec((2, NB, hi), lambda i: (0, i, 0)),
            pl.BlockSpec((NB, hi), lambda i: (i, 0)),
            pl.BlockSpec((NB, hi), lambda i: (i, 0)),
            pl.BlockSpec((1, 2 * hi), lambda i: (0, 0)),
            pl.BlockSpec((2 * hi, D_out), lambda i: (0, 0)),
        ],
        out_specs=pl.BlockSpec((NB, D_out), lambda i: (i, 0)),
        out_shape=jax.ShapeDtypeStruct((N, D_out), _F32),
    )(deg_t, acc, g0, g1, b, W)


def _tc_tail_part(deg_t, acc, g, b, NB):
    N, D = g.shape
    return pl.pallas_call(
        _tc_tail_part_body,
        grid=(N // NB,),
        in_specs=[
            pl.BlockSpec((NB, 2), lambda i: (i, 0)),
            pl.BlockSpec((2, NB, D), lambda i: (0, i, 0)),
            pl.BlockSpec((NB, D), lambda i: (i, 0)),
            pl.BlockSpec((1, D), lambda i: (0, 0)),
        ],
        out_specs=pl.BlockSpec((NB, D), lambda i: (i, 0)),
        out_shape=jax.ShapeDtypeStruct((N, D), _F32),
    )(deg_t, acc, g, b)


def _tc_tail(deg_t, acc, g0, g1, b, NB):
    N, hi = g0.shape
    return pl.pallas_call(
        _tc_tail_body,
        grid=(N // NB,),
        in_specs=[
            pl.BlockSpec((NB, 2), lambda i: (i, 0)),
            pl.BlockSpec((2, NB, hi), lambda i: (0, i, 0)),
            pl.BlockSpec((NB, hi), lambda i: (i, 0)),
            pl.BlockSpec((NB, hi), lambda i: (i, 0)),
            pl.BlockSpec((1, 2 * hi), lambda i: (0, 0)),
        ],
        out_specs=pl.BlockSpec((NB, 2 * hi), lambda i: (i, 0)),
        out_shape=jax.ShapeDtypeStruct((N, 2 * hi), _F32),
    )(deg_t, acc, g0, g1, b)


def kernel(x, edge_index, W1, b1, W2, b2, W3, b3):
    N, _ = x.shape
    E = edge_index.shape[1]
    D_hid = W2.shape[0]
    D_out = W3.shape[1]
    NB = 2000

    src = edge_index[0]
    dst = edge_index[1]
    ones_k = jnp.ones((1000,), _F32)
    zeros_deg = jnp.zeros((N // 16 - 1,), _F32)
    zeros_h = jnp.zeros((N // 16 - 1, D_hid // 2), _F32)
    zeros_o = jnp.zeros((N // 16 - 1, D_out), _F32)

    degp = _make_deg(N, E)(dst, ones_k, zeros_deg)
    deg_t = degp.reshape(2, N).T

    agg_h = _make_agg(N, E, D_hid // 2)
    agg_o = _make_agg_part(N, E, D_out)

    g1a, g1b = _tc_first(deg_t, x, W1, NB)
    acc1 = agg_h(g1a, g1b, src, dst, zeros_h).reshape(2, N, D_hid // 2)
    g2a, g2b = _tc_layer(deg_t, acc1, g1a, g1b, b1.reshape(1, -1), W2, NB)
    acc2 = agg_h(g2a, g2b, src, dst, zeros_h).reshape(2, N, D_hid // 2)
    g3 = _tc_layer1(deg_t, acc2, g2a, g2b, b2.reshape(1, -1), W3, NB)
    acc3 = agg_o(g3, src, dst, zeros_o).reshape(2, N, D_out)
    return _tc_tail_part(deg_t, acc3, g3, b3.reshape(1, -1), NB)


# speedup vs baseline: 1.2550x; 1.0270x over previous
import functools

import jax
import jax.numpy as jnp
from jax import lax
from jax.experimental import pallas as pl
from jax.experimental.pallas import tpu as pltpu
from jax.experimental.pallas import tpu_sc as plsc

_F32 = jnp.float32


def _make_deg(N, E):
    CH = E // 32
    K = 1000
    assert CH % K == 0 and K % 8 == 0
    nchunks = CH // K
    rows = N // 16 - 1
    assert rows % 8 == 0 and rows * 16 + 16 == N
    mesh = plsc.VectorSubcoreMesh(core_axis_name="c", subcore_axis_name="s")

    @functools.partial(
        pl.kernel, mesh=mesh,
        compiler_params=pltpu.CompilerParams(use_tc_tiling_on_sc=False),
        out_type=jax.ShapeDtypeStruct((2 * N,), _F32),
        scratch_types=[
            pltpu.VMEM((K,), jnp.int32),
            pltpu.VMEM((K,), _F32),
            pltpu.VMEM((rows,), _F32),
            pltpu.VMEM_SHARED((N,), _F32),
        ],
    )
    def deg(dst_hbm, ones_hbm, zeros_hbm, out_hbm, idx_d, ones_v, stage_v, sh_deg):
        c = lax.axis_index("c")
        s = lax.axis_index("s")
        wid = s * 2 + c
        pltpu.sync_copy(ones_hbm, ones_v)
        pltpu.sync_copy(zeros_hbm, stage_v)
        pltpu.sync_copy(stage_v, sh_deg.at[pl.ds(s * rows, rows)])

        @pl.when(s == 15)
        def _():
            pltpu.sync_copy(stage_v.at[pl.ds(0, 16)], sh_deg.at[pl.ds(16 * rows, 16)])

        plsc.subcore_barrier()
        e0 = wid * CH
        for k in range(nchunks):
            pltpu.sync_copy(dst_hbm.at[pl.ds(e0 + k * K, K)], idx_d)
            pltpu.sync_copy(ones_v, sh_deg.at[idx_d], add=True)
        plsc.subcore_barrier()
        pltpu.sync_copy(sh_deg.at[pl.ds(s * rows, rows)], stage_v)
        pltpu.sync_copy(stage_v, out_hbm.at[pl.ds(c * N + s * rows, rows)])

        @pl.when(s == 15)
        def _():
            pltpu.sync_copy(sh_deg.at[pl.ds(16 * rows, 16)], stage_v.at[pl.ds(0, 16)])
            pltpu.sync_copy(stage_v.at[pl.ds(0, 16)],
                            out_hbm.at[pl.ds(c * N + 16 * rows, 16)])

    return deg


def _make_agg(N, E, H):
    CH = E // 16
    K = 400
    assert CH % (2 * K) == 0 and K % 8 == 0
    nchunks = CH // K
    rows = N // 16 - 1
    assert rows % 8 == 0 and rows * 16 + 16 == N
    mesh = plsc.VectorSubcoreMesh(core_axis_name="c", subcore_axis_name="s")

    @functools.partial(
        pl.kernel, mesh=mesh,
        compiler_params=pltpu.CompilerParams(use_tc_tiling_on_sc=False),
        out_type=jax.ShapeDtypeStruct((2 * N, H), _F32),
        scratch_types=[
            pltpu.VMEM((5 * K,), jnp.int32),
            [pltpu.VMEM((K,), jnp.int32)] * 5,
            pltpu.VMEM((3 * K, H), _F32),
            pltpu.VMEM_SHARED((N, H), _F32),
            [pltpu.SemaphoreType.DMA] * 3,
            [pltpu.SemaphoreType.DMA] * 3,
            [pltpu.SemaphoreType.DMA] * 5,
        ],
    )
    def agg(g0_hbm, g1_hbm, src_hbm, dst_hbm, zeros_hbm, out_hbm,
            idx_s, idb, rows_v, sh_acc, gsems, ssems, isems):
        c = lax.axis_index("c")
        s = lax.axis_index("s")
        isb = [idx_s.at[pl.ds(i * K, K)] for i in range(5)]
        bufs = [rows_v.at[pl.ds(i * K, K)] for i in range(3)]
        pltpu.sync_copy(zeros_hbm, rows_v.at[pl.ds(0, rows)])
        pltpu.sync_copy(rows_v.at[pl.ds(0, rows)], sh_acc.at[pl.ds(s * rows, rows)])

        @pl.when(s == 15)
        def _():
            pltpu.sync_copy(rows_v.at[pl.ds(0, 16)], sh_acc.at[pl.ds(16 * rows, 16)])

        plsc.subcore_barrier()
        e0 = s * CH

        def edge_loop(g_hbm):
            gh = [None] * 3
            sh = [None] * 3
            ihs = [None] * 5
            ihd = [None] * 5

            def issue_idx(j):
                sl = j % 5
                ihs[sl] = pltpu.async_copy(src_hbm.at[pl.ds(e0 + j * K, K)],
                                           isb[sl], isems[sl])
                ihd[sl] = pltpu.async_copy(dst_hbm.at[pl.ds(e0 + j * K, K)],
                                           idb[sl], isems[sl])

            def issue_gather(j):
                b, sl = j % 3, j % 5
                ihs[sl].wait()
                ihd[sl].wait()
                gh[b] = pltpu.async_copy(g_hbm.at[isb[sl]], bufs[b], gsems[b])

            for j in range(min(3, nchunks)):
                issue_idx(j)
            issue_gather(0)
            if nchunks > 1:
                issue_gather(1)
            for j in range(nchunks):
                b, sl = j % 3, j % 5
                gh[b].wait()
                sh[b] = pltpu.async_copy(bufs[b], sh_acc.at[idb[sl]], ssems[b],
                                         add=True)
                if j + 3 < nchunks:
                    issue_idx(j + 3)
                if j + 2 < nchunks:
                    b2 = (j + 2) % 3
                    if sh[b2] is not None:
                        sh[b2].wait()
                        sh[b2] = None
                    issue_gather(j + 2)
            for b in range(3):
                if sh[b] is not None:
                    sh[b].wait()

        @pl.when(c == 0)
        def _():
            edge_loop(g0_hbm)

        @pl.when(c == 1)
        def _():
            edge_loop(g1_hbm)

        plsc.subcore_barrier()
        pltpu.sync_copy(sh_acc.at[pl.ds(s * rows, rows)], rows_v.at[pl.ds(0, rows)])
        pltpu.sync_copy(rows_v.at[pl.ds(0, rows)],
                        out_hbm.at[pl.ds(c * N + s * rows, rows)])

        @pl.when(s == 15)
        def _():
            pltpu.sync_copy(sh_acc.at[pl.ds(16 * rows, 16)], rows_v.at[pl.ds(0, 16)])
            pltpu.sync_copy(rows_v.at[pl.ds(0, 16)],
                            out_hbm.at[pl.ds(c * N + 16 * rows, 16)])

    return agg


def _make_agg_part(N, E, D):
    CH = E // 32
    K = 400
    assert CH % K == 0 and K % 8 == 0
    nchunks = CH // K
    rows = N // 16 - 1
    assert rows % 8 == 0 and rows * 16 + 16 == N
    mesh = plsc.VectorSubcoreMesh(core_axis_name="c", subcore_axis_name="s")

    @functools.partial(
        pl.kernel, mesh=mesh,
        compiler_params=pltpu.CompilerParams(use_tc_tiling_on_sc=False),
        out_type=jax.ShapeDtypeStruct((2 * N, D), _F32),
        scratch_types=[
            pltpu.VMEM((5 * K,), jnp.int32),
            [pltpu.VMEM((K,), jnp.int32)] * 5,
            pltpu.VMEM((3 * K, D), _F32),
            pltpu.VMEM_SHARED((N, D), _F32),
            [pltpu.SemaphoreType.DMA] * 3,
            [pltpu.SemaphoreType.DMA] * 3,
            [pltpu.SemaphoreType.DMA] * 5,
        ],
    )
    def agg(g_hbm, src_hbm, dst_hbm, zeros_hbm, out_hbm,
            idx_s, idb, rows_v, sh_acc, gsems, ssems, isems):
        c = lax.axis_index("c")
        s = lax.axis_index("s")
        isb = [idx_s.at[pl.ds(i * K, K)] for i in range(5)]
        bufs = [rows_v.at[pl.ds(i * K, K)] for i in range(3)]
        pltpu.sync_copy(zeros_hbm, rows_v.at[pl.ds(0, rows)])
        pltpu.sync_copy(rows_v.at[pl.ds(0, rows)], sh_acc.at[pl.ds(s * rows, rows)])

        @pl.when(s == 15)
        def _():
            pltpu.sync_copy(rows_v.at[pl.ds(0, 16)], sh_acc.at[pl.ds(16 * rows, 16)])

        plsc.subcore_barrier()
        wid = s * 2 + c
        e0 = wid * CH
        gh = [None] * 3
        sh = [None] * 3
        ihs = [None] * 5
        ihd = [None] * 5

        def issue_idx(j):
            sl = j % 5
            ihs[sl] = pltpu.async_copy(src_hbm.at[pl.ds(e0 + j * K, K)],
                                       isb[sl], isems[sl])
            ihd[sl] = pltpu.async_copy(dst_hbm.at[pl.ds(e0 + j * K, K)],
                                       idb[sl], isems[sl])

        def issue_gather(j):
            b, sl = j % 3, j % 5
            ihs[sl].wait()
            ihd[sl].wait()
            gh[b] = pltpu.async_copy(g_hbm.at[isb[sl]], bufs[b], gsems[b])

        for j in range(min(3, nchunks)):
            issue_idx(j)
        issue_gather(0)
        if nchunks > 1:
            issue_gather(1)
        for j in range(nchunks):
            b, sl = j % 3, j % 5
            gh[b].wait()
            sh[b] = pltpu.async_copy(bufs[b], sh_acc.at[idb[sl]], ssems[b],
                                     add=True)
            if j + 3 < nchunks:
                issue_idx(j + 3)
            if j + 2 < nchunks:
                b2 = (j + 2) % 3
                if sh[b2] is not None:
                    sh[b2].wait()
                    sh[b2] = None
                issue_gather(j + 2)
        for b in range(3):
            if sh[b] is not None:
                sh[b].wait()
        plsc.subcore_barrier()
        pltpu.sync_copy(sh_acc.at[pl.ds(s * rows, rows)], rows_v.at[pl.ds(0, rows)])
        pltpu.sync_copy(rows_v.at[pl.ds(0, rows)],
                        out_hbm.at[pl.ds(c * N + s * rows, rows)])

        @pl.when(s == 15)
        def _():
            pltpu.sync_copy(sh_acc.at[pl.ds(16 * rows, 16)], rows_v.at[pl.ds(0, 16)])
            pltpu.sync_copy(rows_v.at[pl.ds(0, 16)],
                            out_hbm.at[pl.ds(c * N + 16 * rows, 16)])

    return agg


_PREC = jax.lax.Precision.HIGHEST


def _dinv(deg_ref):
    d = deg_ref[:, 0] + deg_ref[:, 1] + 1.0
    return lax.rsqrt(d)[:, None]


def _tc_first_body(deg_ref, x_ref, w_ref, o0_ref, o1_ref):
    di = _dinv(deg_ref)
    res = jnp.dot(x_ref[...], w_ref[...],
                  preferred_element_type=_F32, precision=_PREC) * di
    h = res.shape[1] // 2
    o0_ref[...] = res[:, :h]
    o1_ref[...] = res[:, h:]


def _tc_layer_body(deg_ref, acc_ref, g0_ref, g1_ref, b_ref, w_ref,
                   o0_ref, o1_ref):
    di = _dinv(deg_ref)
    w = w_ref[...]
    b = b_ref[...]
    hi = w.shape[0] // 2
    t0 = jnp.maximum((acc_ref[0] + g0_ref[...]) * di + b[:, :hi], 0.0)
    t1 = jnp.maximum((acc_ref[1] + g1_ref[...]) * di + b[:, hi:], 0.0)
    res = (jnp.dot(t0, w[:hi], preferred_element_type=_F32, precision=_PREC)
           + jnp.dot(t1, w[hi:], preferred_element_type=_F32, precision=_PREC)
           ) * di
    ho = res.shape[1] // 2
    o0_ref[...] = res[:, :ho]
    o1_ref[...] = res[:, ho:]


def _tc_layer1_body(deg_ref, acc_ref, g0_ref, g1_ref, b_ref, w_ref, o_ref):
    di = _dinv(deg_ref)
    w = w_ref[...]
    b = b_ref[...]
    hi = w.shape[0] // 2
    t0 = jnp.maximum((acc_ref[0] + g0_ref[...]) * di + b[:, :hi], 0.0)
    t1 = jnp.maximum((acc_ref[1] + g1_ref[...]) * di + b[:, hi:], 0.0)
    o_ref[...] = (jnp.dot(t0, w[:hi], preferred_element_type=_F32,
                          precision=_PREC)
                  + jnp.dot(t1, w[hi:], preferred_element_type=_F32,
                            precision=_PREC)) * di


def _tc_tail_part_body(deg_ref, acc_ref, g_ref, b_ref, o_ref):
    di = _dinv(deg_ref)
    t = (acc_ref[0] + acc_ref[1] + g_ref[...]) * di + b_ref[...]
    t = jnp.maximum(t, 0.0)
    m = jnp.max(t, axis=1, keepdims=True)
    lse = jnp.log(jnp.sum(jnp.exp(t - m), axis=1, keepdims=True)) + m
    o_ref[...] = t - lse


def _tc_tail_body(deg_ref, acc_ref, g0_ref, g1_ref, b_ref, o_ref):
    di = _dinv(deg_ref)
    b = b_ref[...]
    hi = b.shape[1] // 2
    t0 = jnp.maximum((acc_ref[0] + g0_ref[...]) * di + b[:, :hi], 0.0)
    t1 = jnp.maximum((acc_ref[1] + g1_ref[...]) * di + b[:, hi:], 0.0)
    t = jnp.concatenate([t0, t1], axis=1)
    m = jnp.max(t, axis=1, keepdims=True)
    lse = jnp.log(jnp.sum(jnp.exp(t - m), axis=1, keepdims=True)) + m
    o_ref[...] = t - lse


def _tc_first(deg_t, x, W, NB):
    N, D_in = x.shape
    D_out = W.shape[1]
    h = D_out // 2
    return pl.pallas_call(
        _tc_first_body,
        grid=(N // NB,),
        in_specs=[
            pl.BlockSpec((NB, 2), lambda i: (i, 0)),
            pl.BlockSpec((NB, D_in), lambda i: (i, 0)),
            pl.BlockSpec((D_in, D_out), lambda i: (0, 0)),
        ],
        out_specs=[pl.BlockSpec((NB, h), lambda i: (i, 0))] * 2,
        out_shape=[jax.ShapeDtypeStruct((N, h), _F32)] * 2,
    )(deg_t, x, W)


def _tc_layer(deg_t, acc, g0, g1, b, W, NB):
    N, hi = g0.shape
    D_out = W.shape[1]
    ho = D_out // 2
    return pl.pallas_call(
        _tc_layer_body,
        grid=(N // NB,),
        in_specs=[
            pl.BlockSpec((NB, 2), lambda i: (i, 0)),
            pl.BlockSpec((2, NB, hi), lambda i: (0, i, 0)),
            pl.BlockSpec((NB, hi), lambda i: (i, 0)),
            pl.BlockSpec((NB, hi), lambda i: (i, 0)),
            pl.BlockSpec((1, 2 * hi), lambda i: (0, 0)),
            pl.BlockSpec((2 * hi, D_out), lambda i: (0, 0)),
        ],
        out_specs=[pl.BlockSpec((NB, ho), lambda i: (i, 0))] * 2,
        out_shape=[jax.ShapeDtypeStruct((N, ho), _F32)] * 2,
    )(deg_t, acc, g0, g1, b, W)


def _tc_layer1(deg_t, acc, g0, g1, b, W, NB):
    N, hi = g0.shape
    D_out = W.shape[1]
    return pl.pallas_call(
        _tc_layer1_body,
        grid=(N // NB,),
        in_specs=[
            pl.BlockSpec((NB, 2), lambda i: (i, 0)),
            pl.BlockSpec((2, NB, hi), lambda i: (0, i, 0)),
            pl.BlockSpec((NB, hi), lambda i: (i, 0)),
            pl.BlockSpec((NB, hi), lambda i: (i, 0)),
            pl.BlockSpec((1, 2 * hi), lambda i: (0, 0)),
            pl.BlockSpec((2 * hi, D_out), lambda i: (0, 0)),
        ],
        out_specs=pl.BlockSpec((NB, D_out), lambda i: (i, 0)),
        out_shape=jax.ShapeDtypeStruct((N, D_out), _F32),
    )(deg_t, acc, g0, g1, b, W)


def _tc_tail_part(deg_t, acc, g, b, NB):
    N, D = g.shape
    return pl.pallas_call(
        _tc_tail_part_body,
        grid=(N // NB,),
        in_specs=[
            pl.BlockSpec((NB, 2), lambda i: (i, 0)),
            pl.BlockSpec((2, NB, D), lambda i: (0, i, 0)),
            pl.BlockSpec((NB, D), lambda i: (i, 0)),
            pl.BlockSpec((1, D), lambda i: (0, 0)),
        ],
        out_specs=pl.BlockSpec((NB, D), lambda i: (i, 0)),
        out_shape=jax.ShapeDtypeStruct((N, D), _F32),
    )(deg_t, acc, g, b)


def _tc_tail(deg_t, acc, g0, g1, b, NB):
    N, hi = g0.shape
    return pl.pallas_call(
        _tc_tail_body,
        grid=(N // NB,),
        in_specs=[
            pl.BlockSpec((NB, 2), lambda i: (i, 0)),
            pl.BlockSpec((2, NB, hi), lambda i: (0, i, 0)),
            pl.BlockSpec((NB, hi), lambda i: (i, 0)),
            pl.BlockSpec((NB, hi), lambda i: (i, 0)),
            pl.BlockSpec((1, 2 * hi), lambda i: (0, 0)),
        ],
        out_specs=pl.BlockSpec((NB, 2 * hi), lambda i: (i, 0)),
        out_shape=jax.ShapeDtypeStruct((N, 2 * hi), _F32),
    )(deg_t, acc, g0, g1, b)


def kernel(x, edge_index, W1, b1, W2, b2, W3, b3):
    N, _ = x.shape
    E = edge_index.shape[1]
    D_hid = W2.shape[0]
    D_out = W3.shape[1]
    NB = 2000

    src = edge_index[0]
    dst = edge_index[1]
    ones_k = jnp.ones((1000,), _F32)
    zeros_deg = jnp.zeros((N // 16 - 1,), _F32)
    zeros_h = jnp.zeros((N // 16 - 1, D_hid // 2), _F32)
    zeros_o = jnp.zeros((N // 16 - 1, D_out), _F32)

    degp = _make_deg(N, E)(dst, ones_k, zeros_deg)
    deg_t = degp.reshape(2, N).T

    agg_h = _make_agg(N, E, D_hid // 2)
    agg_o = _make_agg_part(N, E, D_out)

    g1a, g1b = _tc_first(deg_t, x, W1, NB)
    acc1 = agg_h(g1a, g1b, src, dst, zeros_h).reshape(2, N, D_hid // 2)
    g2a, g2b = _tc_layer(deg_t, acc1, g1a, g1b, b1.reshape(1, -1), W2, NB)
    acc2 = agg_h(g2a, g2b, src, dst, zeros_h).reshape(2, N, D_hid // 2)
    g3 = _tc_layer1(deg_t, acc2, g2a, g2b, b2.reshape(1, -1), W3, NB)
    acc3 = agg_o(g3, src, dst, zeros_o).reshape(2, N, D_out)
    return _tc_tail_part(deg_t, acc3, g3, b3.reshape(1, -1), NB)
